# Initial kernel scaffold; baseline (speedup 1.0000x reference)
import functools

import jax
import jax.numpy as jnp
from jax import lax
from jax.experimental import pallas as pl
from jax.experimental.pallas import tpu as pltpu
from jax.experimental.pallas import tpu_sc as plsc

_NC = 2
_NS = 16
_NW = _NC * _NS


def _pick_chunk(ept):
    for k in range(128, 7, -8):
        if ept % k == 0:
            return k
    raise ValueError(f"no valid chunk size for {ept} edges per tile")


def _make_deg(n, nchunk, k):
    dd = 16
    rpt = n // _NS
    mesh = plsc.VectorSubcoreMesh(core_axis_name="c", subcore_axis_name="s")

    @functools.partial(
        pl.kernel,
        mesh=mesh,
        out_type=jax.ShapeDtypeStruct((_NC, n, dd), jnp.float32),
        scratch_types=[
            pltpu.VMEM((nchunk, k), jnp.int32),
            pltpu.VMEM((k, dd), jnp.float32),
            pltpu.VMEM_SHARED((n, dd), jnp.float32),
        ],
    )
    def deg(dst_hbm, zero_hbm, out_hbm, dst_v, ones_v, acc):
        cid = lax.axis_index("c")
        sid = lax.axis_index("s")
        wid = sid * _NC + cid
        r0 = sid * rpt
        pltpu.sync_copy(zero_hbm.at[pl.ds(r0, rpt)], acc.at[pl.ds(r0, rpt)])
        pltpu.sync_copy(dst_hbm.at[wid], dst_v)
        for i in range(k):
            ones_v[i, :] = jnp.ones((dd,), jnp.float32)
        plsc.subcore_barrier()

        def body(j, c):
            pltpu.sync_copy(ones_v, acc.at[dst_v.at[j]], add=True)
            return c

        lax.fori_loop(0, nchunk, body, 0)
        plsc.subcore_barrier()
        pltpu.sync_copy(acc.at[pl.ds(r0, rpt)], out_hbm.at[cid, pl.ds(r0, rpt)])

    return deg


def _make_agg(n, d, nchunk, k):
    rpt = n // _NS
    mesh = plsc.VectorSubcoreMesh(core_axis_name="c", subcore_axis_name="s")

    @functools.partial(
        pl.kernel,
        mesh=mesh,
        out_type=jax.ShapeDtypeStruct((_NC, n, d), jnp.float32),
        scratch_types=[
            pltpu.VMEM((nchunk, k), jnp.int32),
            pltpu.VMEM((nchunk, k), jnp.int32),
            pltpu.VMEM((k, d), jnp.float32),
            pltpu.VMEM_SHARED((n, d), jnp.float32),
            pltpu.SemaphoreType.DMA,
        ],
    )
    def agg(src_hbm, dst_hbm, h_hbm, zero_hbm, out_hbm,
            src_v, dst_v, rows_v, acc, sem):
        cid = lax.axis_index("c")
        sid = lax.axis_index("s")
        wid = sid * _NC + cid
        r0 = sid * rpt
        pltpu.sync_copy(zero_hbm.at[pl.ds(r0, rpt)], acc.at[pl.ds(r0, rpt)])
        pltpu.sync_copy(src_hbm.at[wid], src_v)
        pltpu.sync_copy(dst_hbm.at[wid], dst_v)
        plsc.subcore_barrier()

        def body(j, c):
            pltpu.async_copy(h_hbm.at[src_v.at[j]], rows_v, sem).wait()
            pltpu.sync_copy(rows_v, acc.at[dst_v.at[j]], add=True)
            return c

        lax.fori_loop(0, nchunk, body, 0)
        plsc.subcore_barrier()
        pltpu.sync_copy(acc.at[pl.ds(r0, rpt)], out_hbm.at[cid, pl.ds(r0, rpt)])

    return agg


def _dinv_of(p0, p1):
    deg = p0[:, :1] + p1[:, :1] + 1.0
    return lax.rsqrt(deg)


def _mm1_body(x_ref, w_ref, p0_ref, p1_ref, out_ref):
    dinv = _dinv_of(p0_ref, p1_ref)
    out_ref[...] = jnp.dot(x_ref[...], w_ref[...],
                           preferred_element_type=jnp.float32) * dinv


def _mm2_body(a0_ref, a1_ref, hp_ref, p0_ref, p1_ref, b_ref, w_ref, out_ref):
    dinv = _dinv_of(p0_ref, p1_ref)
    h1 = (a0_ref[...] + a1_ref[...] + hp_ref[...]) * dinv + b_ref[...]
    h1 = jnp.maximum(h1, 0.0)
    out_ref[...] = jnp.dot(h1, w_ref[...],
                           preferred_element_type=jnp.float32) * dinv


def _z_body(a0_ref, a1_ref, hp_ref, p0_ref, p1_ref, b_ref, out_ref):
    dinv = _dinv_of(p0_ref, p1_ref)
    out_ref[...] = (a0_ref[...] + a1_ref[...] + hp_ref[...]) * dinv + b_ref[...]


def _dec_body(za_ref, zb_ref, out_ref):
    s = lax.dot_general(za_ref[...], zb_ref[...],
                        (((1,), (1,)), ((), ())),
                        preferred_element_type=jnp.float32)
    out_ref[...] = jax.nn.sigmoid(s)


def kernel(x, edge_index, W1, b1, W2, b2):
    n, din = x.shape
    dhid = W1.shape[1]
    dout = W2.shape[1]
    e = edge_index.shape[1]
    ept = e // _NW
    k = _pick_chunk(ept)
    nchunk = ept // k

    src = edge_index[0].reshape(_NW, nchunk, k)
    dst = edge_index[1].reshape(_NW, nchunk, k)
    z16 = jnp.zeros((n, 16), jnp.float32)
    zhid = jnp.zeros((n, dhid), jnp.float32)
    zout = jnp.zeros((n, dout), jnp.float32)
    b1r = b1.reshape(1, dhid)
    b2r = b2.reshape(1, dout)

    degp = _make_deg(n, nchunk, k)(dst, z16)
    p0, p1 = degp[0], degp[1]

    R = 1000
    grid = (n // R,)
    row = lambda i: (i, 0)
    fixed = lambda i: (0, 0)

    h1p = pl.pallas_call(
        _mm1_body,
        grid=grid,
        in_specs=[
            pl.BlockSpec((R, din), row),
            pl.BlockSpec((din, dhid), fixed),
            pl.BlockSpec((R, 16), row),
            pl.BlockSpec((R, 16), row),
        ],
        out_specs=pl.BlockSpec((R, dhid), row),
        out_shape=jax.ShapeDtypeStruct((n, dhid), jnp.float32),
    )(x, W1, p0, p1)

    agg1 = _make_agg(n, dhid, nchunk, k)(src, dst, h1p, zhid)

    h2p = pl.pallas_call(
        _mm2_body,
        grid=grid,
        in_specs=[
            pl.BlockSpec((R, dhid), row),
            pl.BlockSpec((R, dhid), row),
            pl.BlockSpec((R, dhid), row),
            pl.BlockSpec((R, 16), row),
            pl.BlockSpec((R, 16), row),
            pl.BlockSpec((1, dhid), fixed),
            pl.BlockSpec((dhid, dout), fixed),
        ],
        out_specs=pl.BlockSpec((R, dout), row),
        out_shape=jax.ShapeDtypeStruct((n, dout), jnp.float32),
    )(agg1[0], agg1[1], h1p, p0, p1, b1r, W2)

    agg2 = _make_agg(n, dout, nchunk, k)(src, dst, h2p, zout)

    z = pl.pallas_call(
        _z_body,
        grid=grid,
        in_specs=[
            pl.BlockSpec((R, dout), row),
            pl.BlockSpec((R, dout), row),
            pl.BlockSpec((R, dout), row),
            pl.BlockSpec((R, 16), row),
            pl.BlockSpec((R, 16), row),
            pl.BlockSpec((1, dout), fixed),
        ],
        out_specs=pl.BlockSpec((R, dout), row),
        out_shape=jax.ShapeDtypeStruct((n, dout), jnp.float32),
    )(agg2[0], agg2[1], h2p, p0, p1, b2r)

    BR = 512
    gdec = pl.cdiv(n, BR)
    adj = pl.pallas_call(
        _dec_body,
        grid=(gdec, gdec),
        in_specs=[
            pl.BlockSpec((BR, dout), lambda i, j: (i, 0)),
            pl.BlockSpec((BR, dout), lambda i, j: (j, 0)),
        ],
        out_specs=pl.BlockSpec((BR, BR), lambda i, j: (i, j)),
        out_shape=jax.ShapeDtypeStruct((n, n), jnp.float32),
    )(z, z)

    return (adj, z)


# speedup vs baseline: 10.9460x; 10.9460x over previous
import functools

import jax
import jax.numpy as jnp
from jax import lax
from jax.experimental import pallas as pl
from jax.experimental.pallas import tpu as pltpu
from jax.experimental.pallas import tpu_sc as plsc

_NC = 2
_NS = 16
_NW = _NC * _NS


def _pick_chunk(ept):
    for k in range(128, 7, -8):
        if ept % k == 0:
            return k
    raise ValueError(f"no valid chunk size for {ept} edges per tile")


def _make_deg(n_pad, nchunk, k):
    dd = 16
    rpt = n_pad // _NS
    mesh = plsc.VectorSubcoreMesh(core_axis_name="c", subcore_axis_name="s")

    @functools.partial(
        pl.kernel,
        mesh=mesh,
        out_type=jax.ShapeDtypeStruct((_NC, n_pad, dd), jnp.float32),
        scratch_types=[
            pltpu.VMEM((nchunk, k), jnp.int32),
            pltpu.VMEM((k, dd), jnp.float32),
            pltpu.VMEM_SHARED((n_pad, dd), jnp.float32),
        ],
    )
    def deg(dst_hbm, zero_hbm, out_hbm, dst_v, ones_v, acc):
        cid = lax.axis_index("c")
        sid = lax.axis_index("s")
        wid = sid * _NC + cid
        r0 = sid * rpt
        pltpu.sync_copy(zero_hbm.at[pl.ds(r0, rpt)], acc.at[pl.ds(r0, rpt)])
        pltpu.sync_copy(dst_hbm.at[wid], dst_v)
        for i in range(k):
            ones_v[i, :] = jnp.ones((dd,), jnp.float32)
        plsc.subcore_barrier()

        def body(j, c):
            pltpu.sync_copy(ones_v, acc.at[dst_v.at[j]], add=True)
            return c

        lax.fori_loop(0, nchunk, body, 0)
        plsc.subcore_barrier()
        pltpu.sync_copy(acc.at[pl.ds(r0, rpt)], out_hbm.at[cid, pl.ds(r0, rpt)])

    return deg


def _make_agg(n_pad, d, nchunk, k):
    rpt = n_pad // _NS
    mesh = plsc.VectorSubcoreMesh(core_axis_name="c", subcore_axis_name="s")

    @functools.partial(
        pl.kernel,
        mesh=mesh,
        out_type=jax.ShapeDtypeStruct((_NC, n_pad, d), jnp.float32),
        scratch_types=[
            pltpu.VMEM((nchunk, k), jnp.int32),
            pltpu.VMEM((nchunk, k), jnp.int32),
            pltpu.VMEM((k, d), jnp.float32),
            pltpu.VMEM_SHARED((n_pad, d), jnp.float32),
            pltpu.SemaphoreType.DMA,
        ],
    )
    def agg(src_hbm, dst_hbm, h_hbm, zero_hbm, out_hbm,
            src_v, dst_v, rows_v, acc, sem):
        cid = lax.axis_index("c")
        sid = lax.axis_index("s")
        wid = sid * _NC + cid
        r0 = sid * rpt
        pltpu.sync_copy(zero_hbm.at[pl.ds(r0, rpt)], acc.at[pl.ds(r0, rpt)])
        pltpu.sync_copy(src_hbm.at[wid], src_v)
        pltpu.sync_copy(dst_hbm.at[wid], dst_v)
        plsc.subcore_barrier()

        def body(j, c):
            pltpu.async_copy(h_hbm.at[src_v.at[j]], rows_v, sem).wait()
            pltpu.sync_copy(rows_v, acc.at[dst_v.at[j]], add=True)
            return c

        lax.fori_loop(0, nchunk, body, 0)
        plsc.subcore_barrier()
        pltpu.sync_copy(acc.at[pl.ds(r0, rpt)], out_hbm.at[cid, pl.ds(r0, rpt)])

    return agg


def _dinv_of(p0, p1):
    deg = p0[:, :1] + p1[:, :1] + 1.0
    return lax.rsqrt(deg)


def _mm1_body(x_ref, w_ref, p0_ref, p1_ref, out_ref):
    dinv = _dinv_of(p0_ref, p1_ref)
    out_ref[...] = jnp.dot(x_ref[...], w_ref[...],
                           preferred_element_type=jnp.float32) * dinv


def _mm2_body(a0_ref, a1_ref, hp_ref, p0_ref, p1_ref, b_ref, w_ref, out_ref):
    dinv = _dinv_of(p0_ref, p1_ref)
    h1 = (a0_ref[...] + a1_ref[...] + hp_ref[...]) * dinv + b_ref[...]
    h1 = jnp.maximum(h1, 0.0)
    out_ref[...] = jnp.dot(h1, w_ref[...],
                           preferred_element_type=jnp.float32) * dinv


def _z_body(a0_ref, a1_ref, hp_ref, p0_ref, p1_ref, b_ref, out_ref):
    dinv = _dinv_of(p0_ref, p1_ref)
    out_ref[...] = (a0_ref[...] + a1_ref[...] + hp_ref[...]) * dinv + b_ref[...]


def _dec_body(za_ref, zb_ref, out_ref):
    s = lax.dot_general(za_ref[...], zb_ref[...],
                        (((1,), (1,)), ((), ())),
                        preferred_element_type=jnp.float32)
    out_ref[...] = jax.nn.sigmoid(s)


def kernel(x, edge_index, W1, b1, W2, b2):
    n, din = x.shape
    dhid = W1.shape[1]
    dout = W2.shape[1]
    e = edge_index.shape[1]
    ept = e // _NW
    k = _pick_chunk(ept)
    nchunk = ept // k

    n_pad = ((n + 127) // 128) * 128

    src = edge_index[0].reshape(_NW, nchunk, k)
    dst = edge_index[1].reshape(_NW, nchunk, k)
    z16 = jnp.zeros((n_pad, 16), jnp.float32)
    zhid = jnp.zeros((n_pad, dhid), jnp.float32)
    b1r = b1.reshape(1, dhid)
    b2r = b2.reshape(1, dout)
    d2 = max(dout, 128)
    W2p = jnp.pad(W2, ((0, 0), (0, d2 - dout))) if d2 != dout else W2
    b2p = jnp.pad(b2r, ((0, 0), (0, d2 - dout))) if d2 != dout else b2r

    degp = _make_deg(n_pad, nchunk, k)(dst, z16)
    p0, p1 = degp[0], degp[1]

    R = 1000
    grid = (n // R,)
    row = lambda i: (i, 0)
    fixed = lambda i: (0, 0)

    h1p = pl.pallas_call(
        _mm1_body,
        grid=grid,
        in_specs=[
            pl.BlockSpec((R, din), row),
            pl.BlockSpec((din, dhid), fixed),
            pl.BlockSpec((R, 16), row),
            pl.BlockSpec((R, 16), row),
        ],
        out_specs=pl.BlockSpec((R, dhid), row),
        out_shape=jax.ShapeDtypeStruct((n, dhid), jnp.float32),
    )(x, W1, p0, p1)

    agg1 = _make_agg(n_pad, dhid, nchunk, k)(src, dst, h1p, zhid)

    h2p = pl.pallas_call(
        _mm2_body,
        grid=grid,
        in_specs=[
            pl.BlockSpec((R, dhid), row),
            pl.BlockSpec((R, dhid), row),
            pl.BlockSpec((R, dhid), row),
            pl.BlockSpec((R, 16), row),
            pl.BlockSpec((R, 16), row),
            pl.BlockSpec((1, dhid), fixed),
            pl.BlockSpec((dhid, d2), fixed),
        ],
        out_specs=pl.BlockSpec((R, d2), row),
        out_shape=jax.ShapeDtypeStruct((n, d2), jnp.float32),
    )(agg1[0], agg1[1], h1p, p0, p1, b1r, W2p)

    zo2 = jnp.zeros((n_pad, d2), jnp.float32)
    agg2 = _make_agg(n_pad, d2, nchunk, k)(src, dst, h2p, zo2)

    zfull = pl.pallas_call(
        _z_body,
        grid=grid,
        in_specs=[
            pl.BlockSpec((R, d2), row),
            pl.BlockSpec((R, d2), row),
            pl.BlockSpec((R, d2), row),
            pl.BlockSpec((R, 16), row),
            pl.BlockSpec((R, 16), row),
            pl.BlockSpec((1, d2), fixed),
        ],
        out_specs=pl.BlockSpec((R, d2), row),
        out_shape=jax.ShapeDtypeStruct((n, d2), jnp.float32),
    )(agg2[0], agg2[1], h2p, p0, p1, b2p)

    BR = 512
    gdec = pl.cdiv(n, BR)
    adj = pl.pallas_call(
        _dec_body,
        grid=(gdec, gdec),
        in_specs=[
            pl.BlockSpec((BR, d2), lambda i, j: (i, 0)),
            pl.BlockSpec((BR, d2), lambda i, j: (j, 0)),
        ],
        out_specs=pl.BlockSpec((BR, BR), lambda i, j: (i, j)),
        out_shape=jax.ShapeDtypeStruct((n, n), jnp.float32),
    )(zfull, zfull)

    return (adj, zfull[:, :dout])


# speedup vs baseline: 12.6002x; 1.1511x over previous
import functools

import jax
import jax.numpy as jnp
from jax import lax
from jax.experimental import pallas as pl
from jax.experimental.pallas import tpu as pltpu
from jax.experimental.pallas import tpu_sc as plsc

_NC = 2
_NS = 16
_NW = _NC * _NS


def _pick_chunk(ept):
    for k in range(128, 7, -8):
        if ept % k == 0:
            return k
    raise ValueError(f"no valid chunk size for {ept} edges per tile")


def _make_deg(n_pad, nchunk, k):
    dd = 16
    rpt = n_pad // _NS
    mesh = plsc.VectorSubcoreMesh(core_axis_name="c", subcore_axis_name="s")

    out_t = jax.ShapeDtypeStruct((n_pad, dd), jnp.float32)

    @functools.partial(
        pl.kernel,
        mesh=mesh,
        out_type=[out_t, out_t],
        scratch_types=[
            pltpu.VMEM((nchunk, k), jnp.int32),
            pltpu.VMEM((k, dd), jnp.float32),
            pltpu.VMEM_SHARED((n_pad, dd), jnp.float32),
        ],
    )
    def deg(dst_hbm, zero_hbm, out0_hbm, out1_hbm, dst_v, ones_v, acc):
        cid = lax.axis_index("c")
        sid = lax.axis_index("s")
        wid = sid * _NC + cid
        r0 = sid * rpt
        pltpu.sync_copy(zero_hbm.at[pl.ds(r0, rpt)], acc.at[pl.ds(r0, rpt)])
        pltpu.sync_copy(dst_hbm.at[wid], dst_v)
        for i in range(k):
            ones_v[i, :] = jnp.ones((dd,), jnp.float32)
        plsc.subcore_barrier()

        def body(j, c):
            pltpu.sync_copy(ones_v, acc.at[dst_v.at[j]], add=True)
            return c

        lax.fori_loop(0, nchunk, body, 0)
        plsc.subcore_barrier()

        @pl.when(cid == 0)
        def _():
            pltpu.sync_copy(acc.at[pl.ds(r0, rpt)], out0_hbm.at[pl.ds(r0, rpt)])

        @pl.when(cid == 1)
        def _():
            pltpu.sync_copy(acc.at[pl.ds(r0, rpt)], out1_hbm.at[pl.ds(r0, rpt)])

    return deg


def _make_agg(n_pad, d, nchunk, k):
    rpt = n_pad // _NS
    mesh = plsc.VectorSubcoreMesh(core_axis_name="c", subcore_axis_name="s")

    ept = nchunk * k
    out_t = jax.ShapeDtypeStruct((n_pad, d), jnp.float32)

    @functools.partial(
        pl.kernel,
        mesh=mesh,
        out_type=[out_t, out_t],
        scratch_types=[
            pltpu.VMEM((ept,), jnp.int32),
            pltpu.VMEM((nchunk, k), jnp.int32),
            pltpu.VMEM((k, d), jnp.float32),
            pltpu.VMEM((k, d), jnp.float32),
            pltpu.VMEM_SHARED((n_pad, d), jnp.float32),
            pltpu.SemaphoreType.DMA,
            pltpu.SemaphoreType.DMA,
        ],
    )
    def agg(src_hbm, dst_hbm, h_hbm, zero_hbm, out0_hbm, out1_hbm,
            src_v, dst_v, rows_a, rows_b, acc, sem_a, sem_b):
        cid = lax.axis_index("c")
        sid = lax.axis_index("s")
        wid = sid * _NC + cid
        r0 = sid * rpt
        pltpu.sync_copy(zero_hbm.at[pl.ds(r0, rpt)], acc.at[pl.ds(r0, rpt)])
        pltpu.sync_copy(src_hbm.at[wid], src_v)
        pltpu.sync_copy(dst_hbm.at[wid], dst_v)
        plsc.subcore_barrier()

        def sidx(j):
            return src_v.at[pl.ds(j * k, k)]

        if nchunk % 2 == 1 and nchunk >= 3:
            pltpu.async_copy(h_hbm.at[sidx(0)], rows_a, sem_a)

            def body(i, c):
                j0 = 2 * i
                pltpu.make_async_copy(h_hbm.at[sidx(j0)], rows_a,
                                      sem_a).wait()
                pltpu.async_copy(h_hbm.at[sidx(j0 + 1)], rows_b, sem_b)
                pltpu.sync_copy(rows_a, acc.at[dst_v.at[j0]], add=True)
                pltpu.make_async_copy(h_hbm.at[sidx(j0 + 1)], rows_b,
                                      sem_b).wait()
                pltpu.async_copy(h_hbm.at[sidx(j0 + 2)], rows_a, sem_a)
                pltpu.sync_copy(rows_b, acc.at[dst_v.at[j0 + 1]], add=True)
                return c

            lax.fori_loop(0, (nchunk - 1) // 2, body, 0)
            last = nchunk - 1
            pltpu.make_async_copy(h_hbm.at[sidx(last)], rows_a,
                                  sem_a).wait()
            pltpu.sync_copy(rows_a, acc.at[dst_v.at[last]], add=True)
        else:
            def body(j, c):
                pltpu.async_copy(h_hbm.at[sidx(j)], rows_a, sem_a).wait()
                pltpu.sync_copy(rows_a, acc.at[dst_v.at[j]], add=True)
                return c

            lax.fori_loop(0, nchunk, body, 0)

        plsc.subcore_barrier()

        @pl.when(cid == 0)
        def _():
            pltpu.sync_copy(acc.at[pl.ds(r0, rpt)], out0_hbm.at[pl.ds(r0, rpt)])

        @pl.when(cid == 1)
        def _():
            pltpu.sync_copy(acc.at[pl.ds(r0, rpt)], out1_hbm.at[pl.ds(r0, rpt)])

    return agg


def _dinv_of(p0, p1):
    deg = p0[:, :1] + p1[:, :1] + 1.0
    return lax.rsqrt(deg)


def _mm1_body(x_ref, w_ref, p0_ref, p1_ref, out_ref):
    dinv = _dinv_of(p0_ref, p1_ref)
    out_ref[...] = jnp.dot(x_ref[...], w_ref[...],
                           preferred_element_type=jnp.float32) * dinv


def _mm2_body(a0_ref, a1_ref, hp_ref, p0_ref, p1_ref, b_ref, w_ref, out_ref):
    dinv = _dinv_of(p0_ref, p1_ref)
    h1 = (a0_ref[...] + a1_ref[...] + hp_ref[...]) * dinv + b_ref[...]
    h1 = jnp.maximum(h1, 0.0)
    out_ref[...] = jnp.dot(h1, w_ref[...],
                           preferred_element_type=jnp.float32) * dinv


def _make_z_body(dout):
    def _z_body(a0_ref, a1_ref, hp_ref, p0_ref, p1_ref, b_ref,
                zfull_ref, z_ref):
        dinv = _dinv_of(p0_ref, p1_ref)
        zf = (a0_ref[...] + a1_ref[...] + hp_ref[...]) * dinv + b_ref[...]
        zfull_ref[...] = zf
        z_ref[...] = zf[:, :dout]
    return _z_body


def _dec_body(za_ref, zb_ref, out_ref):
    s = lax.dot_general(za_ref[...], zb_ref[...],
                        (((1,), (1,)), ((), ())),
                        preferred_element_type=jnp.float32)
    out_ref[...] = jax.nn.sigmoid(s)


def kernel(x, edge_index, W1, b1, W2, b2):
    n, din = x.shape
    dhid = W1.shape[1]
    dout = W2.shape[1]
    e = edge_index.shape[1]
    ept = e // _NW
    k = _pick_chunk(ept)
    nchunk = ept // k

    n_pad = ((n + 127) // 128) * 128

    src = edge_index[0].reshape(_NW, ept)
    dst = edge_index[1].reshape(_NW, nchunk, k)
    z16 = jnp.zeros((n_pad, 16), jnp.float32)
    zhid = jnp.zeros((n_pad, dhid), jnp.float32)
    b1r = b1.reshape(1, dhid)
    b2r = b2.reshape(1, dout)
    d2 = max(dout, 128)
    W2p = jnp.pad(W2, ((0, 0), (0, d2 - dout))) if d2 != dout else W2
    b2p = jnp.pad(b2r, ((0, 0), (0, d2 - dout))) if d2 != dout else b2r

    p0, p1 = _make_deg(n_pad, nchunk, k)(dst, z16)

    R = 1000
    grid = (n // R,)
    row = lambda i: (i, 0)
    fixed = lambda i: (0, 0)

    h1p = pl.pallas_call(
        _mm1_body,
        grid=grid,
        in_specs=[
            pl.BlockSpec((R, din), row),
            pl.BlockSpec((din, dhid), fixed),
            pl.BlockSpec((R, 16), row),
            pl.BlockSpec((R, 16), row),
        ],
        out_specs=pl.BlockSpec((R, dhid), row),
        out_shape=jax.ShapeDtypeStruct((n, dhid), jnp.float32),
    )(x, W1, p0, p1)

    a10, a11 = _make_agg(n_pad, dhid, nchunk, k)(src, dst, h1p, zhid)

    h2p = pl.pallas_call(
        _mm2_body,
        grid=grid,
        in_specs=[
            pl.BlockSpec((R, dhid), row),
            pl.BlockSpec((R, dhid), row),
            pl.BlockSpec((R, dhid), row),
            pl.BlockSpec((R, 16), row),
            pl.BlockSpec((R, 16), row),
            pl.BlockSpec((1, dhid), fixed),
            pl.BlockSpec((dhid, d2), fixed),
        ],
        out_specs=pl.BlockSpec((R, d2), row),
        out_shape=jax.ShapeDtypeStruct((n, d2), jnp.float32),
    )(a10, a11, h1p, p0, p1, b1r, W2p)

    zo2 = jnp.zeros((n_pad, d2), jnp.float32)
    a20, a21 = _make_agg(n_pad, d2, nchunk, k)(src, dst, h2p, zo2)

    zfull, z = pl.pallas_call(
        _make_z_body(dout),
        grid=grid,
        in_specs=[
            pl.BlockSpec((R, d2), row),
            pl.BlockSpec((R, d2), row),
            pl.BlockSpec((R, d2), row),
            pl.BlockSpec((R, 16), row),
            pl.BlockSpec((R, 16), row),
            pl.BlockSpec((1, d2), fixed),
        ],
        out_specs=[pl.BlockSpec((R, d2), row), pl.BlockSpec((R, dout), row)],
        out_shape=[jax.ShapeDtypeStruct((n, d2), jnp.float32),
                   jax.ShapeDtypeStruct((n, dout), jnp.float32)],
    )(a20, a21, h2p, p0, p1, b2p)

    BR = 512
    gdec = pl.cdiv(n, BR)
    adj = pl.pallas_call(
        _dec_body,
        grid=(gdec, gdec),
        in_specs=[
            pl.BlockSpec((BR, d2), lambda i, j: (i, 0)),
            pl.BlockSpec((BR, d2), lambda i, j: (j, 0)),
        ],
        out_specs=pl.BlockSpec((BR, BR), lambda i, j: (i, j)),
        out_shape=jax.ShapeDtypeStruct((n, n), jnp.float32),
    )(zfull, zfull)

    return (adj, z)
